# baseline (speedup 1.0000x reference)
import functools

import jax
import jax.numpy as jnp
from jax import lax
from jax.experimental import pallas as pl
from jax.experimental.pallas import tpu as pltpu
from jax.experimental.pallas import tpu_sc as plsc

_NUM_CORES = 2
_NUM_SUBCORES = 16
_NW = _NUM_CORES * _NUM_SUBCORES


def kernel(input_logits, labels, memory, labels_mem):
    batch, d = input_logits.shape
    k = memory.shape[0]
    tail = k - batch

    assert batch % (8 * _NW) == 0 and tail % 8 == 0
    b_per_w = batch // _NW
    t_per_w = -(-(tail // 8) // _NW) * 8

    mesh = plsc.VectorSubcoreMesh(core_axis_name="c", subcore_axis_name="s")

    @functools.partial(
        pl.kernel,
        mesh=mesh,
        out_type=(
            jax.ShapeDtypeStruct((k, d), memory.dtype),
            jax.ShapeDtypeStruct((k,), labels_mem.dtype),
        ),
        scratch_types=[
            pltpu.VMEM((b_per_w,), labels.dtype),
            pltpu.VMEM((t_per_w,), labels_mem.dtype),
            pltpu.SemaphoreType.DMA,
        ],
    )
    def sk(in_hbm, lab_hbm, mem_hbm, labm_hbm, out_mem, out_lab, lv, tv, sem):
        wid = lax.axis_index("s") * _NUM_CORES + lax.axis_index("c")

        ib = wid * b_per_w
        c_in = pltpu.async_copy(
            in_hbm.at[pl.ds(ib, b_per_w)], out_mem.at[pl.ds(ib, b_per_w)], sem
        )
        tb = jnp.minimum(batch + wid * t_per_w, k - t_per_w)
        tb = pl.multiple_of(tb, 8)
        c_mem = pltpu.async_copy(
            mem_hbm.at[pl.ds(tb, t_per_w)], out_mem.at[pl.ds(tb, t_per_w)], sem
        )

        pltpu.sync_copy(lab_hbm.at[pl.ds(ib, b_per_w)], lv)
        pltpu.sync_copy(lv, out_lab.at[pl.ds(ib, b_per_w)])
        pltpu.sync_copy(labm_hbm.at[pl.ds(tb, t_per_w)], tv)
        pltpu.sync_copy(tv, out_lab.at[pl.ds(tb, t_per_w)])

        c_in.wait()
        c_mem.wait()

    new_memory, new_labels_mem = sk(input_logits, labels, memory, labels_mem)
    return (new_memory, new_labels_mem, jnp.array(batch % k, dtype=jnp.int32))
